# Initial kernel scaffold; baseline (speedup 1.0000x reference)
import jax
import jax.numpy as jnp
from jax.experimental import pallas as pl


def kernel(logits, targets):
    raise NotImplementedError("write your pallas kernel here")


# speedup vs baseline: 5.5552x; 5.5552x over previous
import jax
import jax.numpy as jnp
from jax.experimental import pallas as pl
from jax.experimental.pallas import tpu as pltpu

IGNORE = -1


def _loss_kernel(tgt_ref, logits_ref, out_ref, psum_acc, max_acc, tgtl_acc):
    i = pl.program_id(0)
    nsteps = pl.num_programs(0)
    x = logits_ref[...]
    bv = x.shape[1]
    vocab = nsteps * bv

    p = jax.nn.sigmoid(x)
    psum = jnp.sum(p, axis=1, keepdims=True)

    col = jax.lax.broadcasted_iota(jnp.int32, x.shape, 1) + i * bv
    tgt = tgt_ref[...]
    is_t = col == tgt
    max_other = jnp.max(jnp.where(is_t, -jnp.inf, x), axis=1, keepdims=True)
    tgt_logit = jnp.sum(jnp.where(is_t, x, 0.0), axis=1, keepdims=True)

    @pl.when(i == 0)
    def _init():
        psum_acc[...] = psum
        max_acc[...] = max_other
        tgtl_acc[...] = tgt_logit

    @pl.when(i > 0)
    def _update():
        psum_acc[...] += psum
        max_acc[...] = jnp.maximum(max_acc[...], max_other)
        tgtl_acc[...] += tgt_logit

    @pl.when(i == nsteps - 1)
    def _finish():
        mask = (tgt != IGNORE).astype(jnp.float32)
        ps = psum_acc[...]
        mo = max_acc[...]
        tl = tgtl_acc[...]
        loss_simplex = (ps - 1.0) ** 2 / vocab
        loss_margin = jax.nn.softplus(mo - tl)
        p_target = jax.nn.sigmoid(tl)
        loss_brier = (1.0 - p_target) ** 2
        per_row = (loss_simplex + loss_margin + loss_brier) * mask
        total = jnp.sum(per_row, axis=(0, 1), keepdims=True)
        count = jnp.sum(mask, axis=(0, 1), keepdims=True)
        out_ref[...] = jnp.where(count > 0.0,
                                 total / jnp.maximum(count, 1.0),
                                 0.0)


def kernel(logits, targets):
    vocab = logits.shape[-1]
    logits2 = logits.reshape(-1, vocab)
    rows = logits2.shape[0]
    tgt2 = targets.reshape(rows, 1).astype(jnp.int32)

    bv = 3200
    nsteps = vocab // bv
    assert nsteps * bv == vocab

    out = pl.pallas_call(
        _loss_kernel,
        grid=(nsteps,),
        in_specs=[
            pl.BlockSpec((rows, 1), lambda i: (0, 0)),
            pl.BlockSpec((rows, bv), lambda i: (0, i)),
        ],
        out_specs=pl.BlockSpec((1, 1), lambda i: (0, 0)),
        out_shape=jax.ShapeDtypeStruct((1, 1), jnp.float32),
        scratch_shapes=[
            pltpu.VMEM((rows, 1), jnp.float32),
            pltpu.VMEM((rows, 1), jnp.float32),
            pltpu.VMEM((rows, 1), jnp.float32),
        ],
        compiler_params=pltpu.CompilerParams(
            dimension_semantics=("arbitrary",),
        ),
    )(tgt2, logits2)
    return out[0, 0]
